# baseline (speedup 1.0000x reference)
import functools

import numpy as np

import jax
import jax.numpy as jnp
from jax import lax
from jax.experimental import pallas as pl
from jax.experimental.pallas import tpu as pltpu
from jax.experimental.pallas import tpu_sc as plsc

B = 1024
S = 200
H = 64
V = 100000

NC = 2
NS = 16
NW = NC * NS
BPW = B // NW
C0 = 96
C1 = S - C0
HALF = S // 2
NBUF = 4
INV_S = 1.0 / S


def _sc_pool(ids, emb_table):
    mesh = plsc.VectorSubcoreMesh(core_axis_name="c", subcore_axis_name="s")

    @functools.partial(
        pl.kernel,
        out_type=jax.ShapeDtypeStruct((B, H), jnp.float32),
        mesh=mesh,
        scratch_types=[
            pltpu.VMEM((BPW, S), jnp.int32),
            pltpu.VMEM((NBUF, S, H), jnp.bfloat16),
            pltpu.VMEM((BPW, H), jnp.float32),
            pltpu.SemaphoreType.DMA,
            pltpu.SemaphoreType.DMA,
            pltpu.SemaphoreType.DMA,
            pltpu.SemaphoreType.DMA,
        ],
        compiler_params=pltpu.CompilerParams(use_tc_tiling_on_sc=False,
                                             needs_layout_passes=False),
    )
    def pool(ids_hbm, table_hbm, x_hbm, idx_v, rows_v, out_v, s0, s1, s2, s3):
        wid = lax.axis_index("s") * NC + lax.axis_index("c")
        pltpu.sync_copy(ids_hbm.at[pl.ds(wid * BPW, BPW), :], idx_v)
        sems = (s0, s1, s2, s3)

        def fire(r, buf):
            return [
                pltpu.async_copy(
                    table_hbm.at[idx_v.at[r, pl.ds(0, C0)]],
                    rows_v.at[buf, pl.ds(0, C0)],
                    sems[buf],
                ),
                pltpu.async_copy(
                    table_hbm.at[idx_v.at[r, pl.ds(C0, C1)]],
                    rows_v.at[buf, pl.ds(C0, C1)],
                    sems[buf],
                ),
            ]

        def accum_store(r, buf):
            zero = jnp.zeros((16,), jnp.float32)
            ngrp = H // 32

            @plsc.parallel_loop(0, HALF, 1, unroll=4,
                                carry=(zero,) * (2 * 2 * ngrp))
            def accs(s, a):
                a = list(a)
                for half in range(2):
                    for g in range(ngrp):
                        v = rows_v[buf, s + HALF * half, pl.ds(32 * g, 32)]
                        ea, eb = plsc.unpack(
                            v, format=plsc.PackFormat.INTERLEAVED,
                            preferred_element_type=jnp.float32)
                        k = (half * ngrp + g) * 2
                        a[k] = a[k] + ea
                        a[k + 1] = a[k + 1] + eb
                return tuple(a)

            for g in range(ngrp):
                for j in range(2):
                    tot = accs[2 * g + j] + accs[(ngrp + g) * 2 + j]
                    out_v[r, pl.ds(32 * g + 16 * j, 16)] = tot * INV_S

        pending = {r: fire(r, r) for r in range(NBUF - 1)}
        for r in range(BPW):
            buf = r % NBUF
            if r + NBUF - 1 < BPW:
                pending[r + NBUF - 1] = fire(r + NBUF - 1, (r + NBUF - 1) % NBUF)
            for d in pending.pop(r):
                d.wait()
            accum_store(r, buf)

        pltpu.sync_copy(out_v, x_hbm.at[pl.ds(wid * BPW, BPW), :])

    return pool(ids, emb_table)


TILE_V = 2048


def _mm_body(w_ref, x_ref, b_ref, o_ref):
    o_ref[...] = (
        lax.dot_general(
            w_ref[...], x_ref[...], (((0,), (1,)), ((), ())),
            preferred_element_type=jnp.float32,
        )
        + b_ref[...].T
    )


def _tc_project(x, W, b2):
    grid = (pl.cdiv(V, TILE_V),)
    out = pl.pallas_call(
        _mm_body,
        grid=grid,
        in_specs=[
            pl.BlockSpec((H, TILE_V), lambda i: (0, i)),
            pl.BlockSpec((B, H), lambda i: (0, 0)),
            pl.BlockSpec((1, TILE_V), lambda i: (0, i)),
        ],
        out_specs=pl.BlockSpec((TILE_V, B), lambda i: (i, 0)),
        out_shape=jax.ShapeDtypeStruct((V, B), jnp.float32),
    )(W, x, b2)
    return out.T


_J = np.arange(H)
_INV = 32 * (_J // 32) + 16 * (_J % 2) + (_J % 32) // 2


def kernel(input_ids, emb_table, W, b):
    x_stored = _sc_pool(input_ids.astype(jnp.int32),
                        emb_table.astype(jnp.bfloat16))
    x = x_stored[:, _INV]
    return _tc_project(x, W, b.reshape(1, V))


# speedup vs baseline: 1.0271x; 1.0271x over previous
import functools

import jax
import jax.numpy as jnp
from jax import lax
from jax.experimental import pallas as pl
from jax.experimental.pallas import tpu as pltpu
from jax.experimental.pallas import tpu_sc as plsc

B = 1024
S = 200
H = 64
V = 100000

NC = 2
NS = 16
NW = NC * NS
BPW = B // NW
C0 = 96
C1 = S - C0
HALF = S // 2
NBUF = 2
INV_S = 1.0 / S
HP = 128


def _sc_pool(ids, emb_table):
    mesh = plsc.VectorSubcoreMesh(core_axis_name="c", subcore_axis_name="s")

    @functools.partial(
        pl.kernel,
        out_type=jax.ShapeDtypeStruct((B, H), jnp.float32),
        name="sc_pool",
        mesh=mesh,
        scratch_types=[
            pltpu.VMEM((BPW, S), jnp.int32),
            pltpu.VMEM((NBUF, S, HP), jnp.float32),
            pltpu.VMEM((BPW, H), jnp.float32),
            pltpu.SemaphoreType.DMA,
            pltpu.SemaphoreType.DMA,
            pltpu.SemaphoreType.DMA,
            pltpu.SemaphoreType.DMA,
        ],
        compiler_params=pltpu.CompilerParams(use_tc_tiling_on_sc=False),
    )
    def pool(ids_hbm, table_hbm, x_hbm, idx_v, rows_v, out_v, s0, s1, s2, s3):
        wid = lax.axis_index("s") * NC + lax.axis_index("c")
        pltpu.sync_copy(ids_hbm.at[pl.ds(wid * BPW, BPW), :], idx_v)
        sems = (s0, s1, s2, s3)

        def fire(r, buf):
            return [
                pltpu.async_copy(
                    table_hbm.at[idx_v.at[r, pl.ds(0, C0)]],
                    rows_v.at[buf, pl.ds(0, C0)],
                    sems[buf],
                ),
                pltpu.async_copy(
                    table_hbm.at[idx_v.at[r, pl.ds(C0, C1)]],
                    rows_v.at[buf, pl.ds(C0, C1)],
                    sems[buf],
                ),
            ]

        def accum_store(r, buf):
            zero = jnp.zeros((16,), jnp.float32)
            ngrp = H // 16

            @plsc.parallel_loop(0, HALF, 1, unroll=4,
                                carry=(zero,) * (2 * ngrp))
            def accs(s, a):
                a = list(a)
                for half in range(2):
                    for g in range(ngrp):
                        k = half * ngrp + g
                        a[k] = a[k] + rows_v[buf, s + HALF * half,
                                             pl.ds(16 * g, 16)]
                return tuple(a)

            for g in range(ngrp):
                out_v[r, pl.ds(16 * g, 16)] = (accs[g] + accs[ngrp + g]) * INV_S

        pending = {r: fire(r, r) for r in range(NBUF - 1)}
        for r in range(BPW):
            buf = r % NBUF
            if r + NBUF - 1 < BPW:
                pending[r + NBUF - 1] = fire(r + NBUF - 1, (r + NBUF - 1) % NBUF)
            for d in pending.pop(r):
                d.wait()
            accum_store(r, buf)

        pltpu.sync_copy(out_v, x_hbm.at[pl.ds(wid * BPW, BPW), :])

    return pool(ids, emb_table)


TILE_V = 2048


def _mm_body(w_ref, x_ref, b_ref, o_ref):
    o_ref[...] = (
        lax.dot_general(
            w_ref[...], x_ref[...], (((0,), (1,)), ((), ())),
            preferred_element_type=jnp.float32,
        )
        + b_ref[...].T
    )


def _tc_project(x, W, b2):
    grid = (pl.cdiv(V, TILE_V),)
    out = pl.pallas_call(
        _mm_body,
        grid=grid,
        in_specs=[
            pl.BlockSpec((H, TILE_V), lambda i: (0, i)),
            pl.BlockSpec((B, H), lambda i: (0, 0)),
            pl.BlockSpec((1, TILE_V), lambda i: (0, i)),
        ],
        out_specs=pl.BlockSpec((TILE_V, B), lambda i: (i, 0)),
        out_shape=jax.ShapeDtypeStruct((V, B), jnp.float32),
    )(W, x, b2)
    return out.T


def kernel(input_ids, emb_table, W, b):
    emb_pad = jnp.pad(emb_table, ((0, 0), (0, HP - H)))
    x = _sc_pool(input_ids.astype(jnp.int32), emb_pad)
    return _tc_project(x, W, b.reshape(1, V))


# speedup vs baseline: 1.1215x; 1.0919x over previous
import functools

import jax
import jax.numpy as jnp
from jax import lax
from jax.experimental import pallas as pl
from jax.experimental.pallas import tpu as pltpu
from jax.experimental.pallas import tpu_sc as plsc

B = 1024
S = 200
H = 64
V = 100000

NC = 2
NS = 16
NW = NC * NS
BPW = B // NW
C0 = 96
C1 = S - C0
HALF = S // 2
NBUF = 4
INV_S = 1.0 / S


def _sc_pool(ids, emb_table):
    mesh = plsc.VectorSubcoreMesh(core_axis_name="c", subcore_axis_name="s")

    @functools.partial(
        pl.kernel,
        out_type=jax.ShapeDtypeStruct((B, H), jnp.float32),
        mesh=mesh,
        scratch_types=[
            pltpu.VMEM((BPW, S), jnp.int32),
            pltpu.VMEM((NBUF, S, H), jnp.float32),
            pltpu.VMEM((BPW, H), jnp.float32),
            pltpu.SemaphoreType.DMA,
            pltpu.SemaphoreType.DMA,
            pltpu.SemaphoreType.DMA,
            pltpu.SemaphoreType.DMA,
        ],
        compiler_params=pltpu.CompilerParams(use_tc_tiling_on_sc=False),
    )
    def pool(ids_hbm, table_hbm, x_hbm, idx_v, rows_v, out_v, s0, s1, s2, s3):
        wid = lax.axis_index("s") * NC + lax.axis_index("c")
        pltpu.sync_copy(ids_hbm.at[pl.ds(wid * BPW, BPW), :], idx_v)
        sems = (s0, s1, s2, s3)

        def fire(r, buf):
            return [
                pltpu.async_copy(
                    table_hbm.at[idx_v.at[r, pl.ds(0, C0)]],
                    rows_v.at[buf, pl.ds(0, C0)],
                    sems[buf],
                ),
                pltpu.async_copy(
                    table_hbm.at[idx_v.at[r, pl.ds(C0, C1)]],
                    rows_v.at[buf, pl.ds(C0, C1)],
                    sems[buf],
                ),
            ]

        def accum_store(r, buf):
            zero = jnp.zeros((16,), jnp.float32)
            ngrp = H // 16

            @plsc.parallel_loop(0, HALF, 1, unroll=4,
                                carry=(zero,) * (2 * ngrp))
            def accs(s, a):
                a = list(a)
                for half in range(2):
                    for g in range(ngrp):
                        k = half * ngrp + g
                        a[k] = a[k] + rows_v[buf, s + HALF * half,
                                             pl.ds(16 * g, 16)]
                return tuple(a)

            for g in range(ngrp):
                out_v[r, pl.ds(16 * g, 16)] = (accs[g] + accs[ngrp + g]) * INV_S

        pending = {r: fire(r, r) for r in range(NBUF - 1)}
        for r in range(BPW):
            buf = r % NBUF
            if r + NBUF - 1 < BPW:
                pending[r + NBUF - 1] = fire(r + NBUF - 1, (r + NBUF - 1) % NBUF)
            for d in pending.pop(r):
                d.wait()
            accum_store(r, buf)

        pltpu.sync_copy(out_v, x_hbm.at[pl.ds(wid * BPW, BPW), :])

    return pool(ids, emb_table)


TILE_V = 2048


def _mm_body(w_ref, x_ref, b_ref, o_ref):
    o_ref[...] = (
        lax.dot_general(
            w_ref[...], x_ref[...], (((0,), (1,)), ((), ())),
            preferred_element_type=jnp.float32,
        )
        + b_ref[...].T
    )


def _tc_project(x, W, b2):
    grid = (pl.cdiv(V, TILE_V),)
    out = pl.pallas_call(
        _mm_body,
        grid=grid,
        in_specs=[
            pl.BlockSpec((H, TILE_V), lambda i: (0, i)),
            pl.BlockSpec((B, H), lambda i: (0, 0)),
            pl.BlockSpec((1, TILE_V), lambda i: (0, i)),
        ],
        out_specs=pl.BlockSpec((TILE_V, B), lambda i: (i, 0)),
        out_shape=jax.ShapeDtypeStruct((V, B), jnp.float32),
    )(W, x, b2)
    return out.T


def kernel(input_ids, emb_table, W, b):
    x = _sc_pool(input_ids.astype(jnp.int32), emb_table)
    return _tc_project(x, W, b.reshape(1, V))
